# Initial kernel scaffold; baseline (speedup 1.0000x reference)
import functools

import jax
import jax.numpy as jnp
from jax import lax
from jax.experimental import pallas as pl
from jax.experimental.pallas import tpu as pltpu
from jax.experimental.pallas import tpu_sc as plsc

N_NODES = 10000
N_EDGES = 320000
D_FEAT = 128
EMBED_DIM = 128
BATCH = 4096
NUM_FIELDS = 26

NC = 2
NS = 16
NW = NC * NS
L = 16

CH = 79
EPW = CH * 128
EPAD = NW * EPW
NP = 10240
RPT = NP // NS
DUMMY = N_NODES + 16
XROWS = BATCH * NUM_FIELDS // 128
XPW = XROWS // NW

_MESH = plsc.VectorSubcoreMesh(core_axis_name="c", subcore_axis_name="s")


@functools.partial(
    pl.kernel,
    out_type=jax.ShapeDtypeStruct((NW, NP), jnp.float32),
    mesh=_MESH,
    scratch_types=[
        pltpu.VMEM((CH, 128), jnp.int32),
        pltpu.VMEM((NP,), jnp.float32),
    ],
)
def _sc_degree(dst_hbm, out_hbm, idx_v, deg_v):
    c = lax.axis_index("c")
    s = lax.axis_index("s")
    wid = c * NS + s
    pltpu.sync_copy(dst_hbm.at[pl.ds(wid * CH, CH)], idx_v)

    zeros = jnp.zeros((L,), jnp.float32)

    def _zero(i, carry):
        deg_v[pl.ds(i * L, L)] = zeros
        return carry

    lax.fori_loop(0, NP // L, _zero, 0)

    ones = jnp.ones((L,), jnp.float32)

    def _chunk(j, carry):
        def _sub(k, carry2):
            idx16 = idx_v[j, pl.ds(k * L, L)]
            plsc.addupdate_scatter(deg_v, [idx16], ones)
            return carry2

        return lax.fori_loop(0, 128 // L, _sub, carry)

    lax.fori_loop(0, CH, _chunk, 0)
    pltpu.sync_copy(deg_v, out_hbm.at[wid])


def _tc_prep_body(feat_ref, w_ref, degp_ref, y_ref):
    xw = jnp.dot(feat_ref[...], w_ref[...], preferred_element_type=jnp.float32)
    deg = jnp.sum(degp_ref[...], axis=0) + 1.0
    dinv = lax.rsqrt(deg)[:, None]
    y_ref[...] = xw * dinv


_tc_prep = pl.pallas_call(
    _tc_prep_body,
    grid=(10,),
    in_specs=[
        pl.BlockSpec((1000, D_FEAT), lambda i: (i, 0)),
        pl.BlockSpec((D_FEAT, EMBED_DIM), lambda i: (0, 0)),
        pl.BlockSpec((NW, 1000), lambda i: (0, i)),
    ],
    out_specs=pl.BlockSpec((1000, EMBED_DIM), lambda i: (i, 0)),
    out_shape=jax.ShapeDtypeStruct((N_NODES, EMBED_DIM), jnp.float32),
)


@functools.partial(
    pl.kernel,
    out_type=jax.ShapeDtypeStruct((NC, NP, EMBED_DIM), jnp.float32),
    mesh=_MESH,
    scratch_types=[
        pltpu.VMEM((CH, 128), jnp.int32),
        pltpu.VMEM((CH, 128), jnp.int32),
        pltpu.VMEM((128, EMBED_DIM), jnp.float32),
        pltpu.VMEM_SHARED((NP, EMBED_DIM), jnp.float32),
        pltpu.SemaphoreType.DMA,
    ],
)
def _sc_scatter(src_hbm, dst_hbm, y_hbm, zeros_hbm, out_hbm,
                src_v, dst_v, rows_v, acc_sh, sem):
    c = lax.axis_index("c")
    s = lax.axis_index("s")
    wid = c * NS + s
    pltpu.sync_copy(src_hbm.at[pl.ds(wid * CH, CH)], src_v)
    pltpu.sync_copy(dst_hbm.at[pl.ds(wid * CH, CH)], dst_v)
    pltpu.sync_copy(zeros_hbm, acc_sh.at[pl.ds(s * RPT, RPT)])
    plsc.subcore_barrier()

    def _chunk(j, carry):
        pltpu.async_copy(y_hbm.at[src_v.at[j]], rows_v, sem).wait()
        pltpu.sync_copy(rows_v, acc_sh.at[dst_v.at[j]], add=True)
        return carry

    lax.fori_loop(0, CH, _chunk, 0)
    plsc.subcore_barrier()
    pltpu.sync_copy(acc_sh.at[pl.ds(s * RPT, RPT)],
                    out_hbm.at[c, pl.ds(s * RPT, RPT)])


def _tc_combine_body(acc_ref, y_ref, degp_ref, b_ref, out_ref):
    acc = acc_ref[0] + acc_ref[1]
    deg = jnp.sum(degp_ref[...], axis=0) + 1.0
    dinv = lax.rsqrt(deg)[:, None]
    out_ref[...] = dinv * (acc + y_ref[...]) + b_ref[...][None, :]


_tc_combine = pl.pallas_call(
    _tc_combine_body,
    grid=(10,),
    in_specs=[
        pl.BlockSpec((NC, 1000, EMBED_DIM), lambda i: (0, i, 0)),
        pl.BlockSpec((1000, EMBED_DIM), lambda i: (i, 0)),
        pl.BlockSpec((NW, 1000), lambda i: (0, i)),
        pl.BlockSpec((EMBED_DIM,), lambda i: (0,)),
    ],
    out_specs=pl.BlockSpec((1000, EMBED_DIM), lambda i: (i, 0)),
    out_shape=jax.ShapeDtypeStruct((N_NODES, EMBED_DIM), jnp.float32),
)


@functools.partial(
    pl.kernel,
    out_type=jax.ShapeDtypeStruct((BATCH * NUM_FIELDS, EMBED_DIM),
                                  jnp.float32),
    mesh=_MESH,
    scratch_types=[
        pltpu.VMEM((XPW, 128), jnp.int32),
        pltpu.VMEM((128, EMBED_DIM), jnp.float32),
        pltpu.SemaphoreType.DMA,
    ],
)
def _sc_gather(emb_hbm, x_hbm, out_hbm, idx_v, rows_v, sem):
    c = lax.axis_index("c")
    s = lax.axis_index("s")
    wid = c * NS + s
    pltpu.sync_copy(x_hbm.at[pl.ds(wid * XPW, XPW)], idx_v)

    def _chunk(j, carry):
        pltpu.async_copy(emb_hbm.at[idx_v.at[j]], rows_v, sem).wait()
        pltpu.sync_copy(rows_v,
                        out_hbm.at[pl.ds(wid * XPW * 128 + j * 128, 128)])
        return carry

    lax.fori_loop(0, XPW, _chunk, 0)


def kernel(x, features, edge_index, W, b):
    src = edge_index[0].astype(jnp.int32)
    dst = edge_index[1].astype(jnp.int32)
    npad = EPAD - N_EDGES
    src2d = jnp.concatenate(
        [src, jnp.zeros((npad,), jnp.int32)]).reshape(NW * CH, 128)
    dst2d = jnp.concatenate(
        [dst, jnp.full((npad,), DUMMY, jnp.int32)]).reshape(NW * CH, 128)
    x2d = x.astype(jnp.int32).reshape(XROWS, 128)
    zeros_blk = jnp.zeros((RPT, EMBED_DIM), jnp.float32)

    degp = _sc_degree(dst2d)
    y = _tc_prep(features, W, degp)
    acc = _sc_scatter(src2d, dst2d, y, zeros_blk)
    emb = _tc_combine(acc, y, degp, b)
    out = _sc_gather(emb, x2d)
    return out.reshape(BATCH, NUM_FIELDS, EMBED_DIM)


# speedup vs baseline: 13.5735x; 13.5735x over previous
import functools

import jax
import jax.numpy as jnp
from jax import lax
from jax.experimental import pallas as pl
from jax.experimental.pallas import tpu as pltpu
from jax.experimental.pallas import tpu_sc as plsc

N_NODES = 10000
N_EDGES = 320000
D_FEAT = 128
EMBED_DIM = 128
BATCH = 4096
NUM_FIELDS = 26

NC = 2
NS = 16
NW = NC * NS
L = 16

CH = 79
EPW = CH * 128
EPAD = NW * EPW
NP = 10240
RPT = NP // NS
DUMMY = N_NODES + 16
XROWS = BATCH * NUM_FIELDS // 128
XPW = XROWS // NW

_MESH = plsc.VectorSubcoreMesh(core_axis_name="c", subcore_axis_name="s")


@functools.partial(
    pl.kernel,
    out_type=jax.ShapeDtypeStruct((NW * NP,), jnp.float32),
    mesh=_MESH,
    scratch_types=[
        pltpu.VMEM((CH, 128), jnp.int32),
        pltpu.VMEM((NP,), jnp.float32),
    ],
    compiler_params=pltpu.CompilerParams(needs_layout_passes=False),
)
def _sc_degree(dst_hbm, out_hbm, idx_v, deg_v):
    c = lax.axis_index("c")
    s = lax.axis_index("s")
    wid = c * NS + s
    pltpu.sync_copy(dst_hbm.at[wid], idx_v)

    zeros = jnp.zeros((L,), jnp.float32)

    def _zero(i, carry):
        deg_v[pl.ds(i * L, L)] = zeros
        return carry

    lax.fori_loop(0, NP // L, _zero, 0)

    ones = jnp.ones((L,), jnp.float32)

    def _chunk(j, carry):
        def _sub(k, carry2):
            idx16 = idx_v[j, pl.ds(k * L, L)]
            plsc.addupdate_scatter(deg_v, [idx16], ones)
            return carry2

        return lax.fori_loop(0, 128 // L, _sub, carry)

    lax.fori_loop(0, CH, _chunk, 0)
    pltpu.sync_copy(deg_v, out_hbm.at[pl.ds(wid * NP, NP)])


def _dinv_col(degp_blk):
    ones = jnp.ones((NW, 1), jnp.float32)
    deg = lax.dot_general(degp_blk, ones, (((0,), (0,)), ((), ())),
                          preferred_element_type=jnp.float32) + 1.0
    return lax.rsqrt(deg)


def _tc_prep_body(feat_ref, w_ref, degp_ref, y_ref):
    xw = jnp.dot(feat_ref[...], w_ref[...], preferred_element_type=jnp.float32)
    y_ref[...] = xw * _dinv_col(degp_ref[...])


_TCR = 1024

_tc_prep = pl.pallas_call(
    _tc_prep_body,
    grid=(NP // _TCR,),
    in_specs=[
        pl.BlockSpec((_TCR, D_FEAT), lambda i: (i, 0)),
        pl.BlockSpec((D_FEAT, EMBED_DIM), lambda i: (0, 0)),
        pl.BlockSpec((NW, _TCR), lambda i: (0, i)),
    ],
    out_specs=pl.BlockSpec((_TCR, EMBED_DIM), lambda i: (i, 0)),
    out_shape=jax.ShapeDtypeStruct((NP, EMBED_DIM), jnp.float32),
)


@functools.partial(
    pl.kernel,
    out_type=jax.ShapeDtypeStruct((NC, NP, EMBED_DIM), jnp.float32),
    mesh=_MESH,
    scratch_types=[
        pltpu.VMEM((CH, 128), jnp.int32),
        pltpu.VMEM((CH, 128), jnp.int32),
        pltpu.VMEM((128, EMBED_DIM), jnp.float32),
        pltpu.VMEM_SHARED((NP, EMBED_DIM), jnp.float32),
        pltpu.SemaphoreType.DMA,
    ],
)
def _sc_scatter(src_hbm, dst_hbm, y_hbm, zeros_hbm, out_hbm,
                src_v, dst_v, rows_v, acc_sh, sem):
    c = lax.axis_index("c")
    s = lax.axis_index("s")
    wid = c * NS + s
    pltpu.sync_copy(src_hbm.at[wid], src_v)
    pltpu.sync_copy(dst_hbm.at[wid], dst_v)
    pltpu.sync_copy(zeros_hbm, acc_sh.at[pl.ds(s * RPT, RPT)])
    plsc.subcore_barrier()

    def _chunk(j, carry):
        pltpu.async_copy(y_hbm.at[src_v.at[j]], rows_v, sem).wait()
        pltpu.sync_copy(rows_v, acc_sh.at[dst_v.at[j]], add=True)
        return carry

    lax.fori_loop(0, CH, _chunk, 0)
    plsc.subcore_barrier()
    pltpu.sync_copy(acc_sh.at[pl.ds(s * RPT, RPT)],
                    out_hbm.at[c, pl.ds(s * RPT, RPT)])


def _tc_combine_body(acc_ref, y_ref, degp_ref, b_ref, out_ref):
    acc = acc_ref[0] + acc_ref[1]
    dinv = _dinv_col(degp_ref[...])
    out_ref[...] = dinv * (acc + y_ref[...]) + b_ref[...][None, :]


_tc_combine = pl.pallas_call(
    _tc_combine_body,
    grid=(NP // _TCR,),
    in_specs=[
        pl.BlockSpec((NC, _TCR, EMBED_DIM), lambda i: (0, i, 0)),
        pl.BlockSpec((_TCR, EMBED_DIM), lambda i: (i, 0)),
        pl.BlockSpec((NW, _TCR), lambda i: (0, i)),
        pl.BlockSpec((EMBED_DIM,), lambda i: (0,)),
    ],
    out_specs=pl.BlockSpec((_TCR, EMBED_DIM), lambda i: (i, 0)),
    out_shape=jax.ShapeDtypeStruct((NP, EMBED_DIM), jnp.float32),
)


@functools.partial(
    pl.kernel,
    out_type=jax.ShapeDtypeStruct((BATCH * NUM_FIELDS, EMBED_DIM),
                                  jnp.float32),
    mesh=_MESH,
    scratch_types=[
        pltpu.VMEM((XPW, 128), jnp.int32),
        pltpu.VMEM((128, EMBED_DIM), jnp.float32),
        pltpu.SemaphoreType.DMA,
    ],
)
def _sc_gather(emb_hbm, x_hbm, out_hbm, idx_v, rows_v, sem):
    c = lax.axis_index("c")
    s = lax.axis_index("s")
    wid = c * NS + s
    pltpu.sync_copy(x_hbm.at[wid], idx_v)

    def _chunk(j, carry):
        pltpu.async_copy(emb_hbm.at[idx_v.at[j]], rows_v, sem).wait()
        pltpu.sync_copy(rows_v,
                        out_hbm.at[pl.ds(wid * XPW * 128 + j * 128, 128)])
        return carry

    lax.fori_loop(0, XPW, _chunk, 0)


def kernel(x, features, edge_index, W, b):
    src = edge_index[0].astype(jnp.int32)
    dst = edge_index[1].astype(jnp.int32)
    npad = EPAD - N_EDGES
    src2d = jnp.concatenate(
        [src, jnp.zeros((npad,), jnp.int32)]).reshape(NW, CH, 128)
    dst2d = jnp.concatenate(
        [dst, jnp.full((npad,), DUMMY, jnp.int32)]).reshape(NW, CH, 128)
    x2d = x.astype(jnp.int32).reshape(NW, XPW, 128)
    zeros_blk = jnp.zeros((RPT, EMBED_DIM), jnp.float32)
    feat_pad = jnp.concatenate(
        [features, jnp.zeros((NP - N_NODES, D_FEAT), jnp.float32)])

    degp = _sc_degree(dst2d).reshape(NW, NP)
    y = _tc_prep(feat_pad, W, degp)
    acc = _sc_scatter(src2d, dst2d, y, zeros_blk)
    emb = _tc_combine(acc, y, degp, b)
    out = _sc_gather(emb, x2d)
    return out.reshape(BATCH, NUM_FIELDS, EMBED_DIM)
